# baseline (speedup 1.0000x reference)
import jax
import jax.numpy as jnp
from jax import lax
from jax.experimental import pallas as pl
from jax.experimental.pallas import tpu as pltpu
from jax.experimental.pallas import tpu_sc as plsc

B = 1024
L = 200
N = B * L
WORD_DIM = 128
SMALL_DIM = 16
OUT_DIM = WORD_DIM + 3 * SMALL_DIM

NC = 2
NS = 16
NW = NC * NS
N_PER_W = N // NW
CHUNK = 128
M = N_PER_W // CHUNK
K = 4
D = 2


def _sc_body(idx4_hbm,
             word_tbl, tag_tbl, pos1_tbl, pos2_tbl,
             out_hbm,
             idxb, wb, tb, p1b, p2b, gsems, wsems):
  wid = lax.axis_index("s") * NC + lax.axis_index("c")
  wbase = wid * N_PER_W

  def fire_gathers(t, b):
    base = wbase + t * CHUNK
    pltpu.sync_copy(idx4_hbm.at[:, pl.ds(base, CHUNK)], idxb[b])
    pltpu.async_copy(word_tbl.at[idxb[b].at[0]], wb[b], gsems[b])
    pltpu.async_copy(tag_tbl.at[idxb[b].at[1]], tb[b], gsems[b])
    pltpu.async_copy(pos1_tbl.at[idxb[b].at[2]], p1b[b], gsems[b])
    pltpu.async_copy(pos2_tbl.at[idxb[b].at[3]], p2b[b], gsems[b])

  def wait_gathers(b):
    pltpu.make_async_copy(word_tbl.at[idxb[b].at[0]], wb[b], gsems[b]).wait()
    pltpu.make_async_copy(tag_tbl.at[idxb[b].at[1]], tb[b], gsems[b]).wait()
    pltpu.make_async_copy(pos1_tbl.at[idxb[b].at[2]], p1b[b], gsems[b]).wait()
    pltpu.make_async_copy(pos2_tbl.at[idxb[b].at[3]], p2b[b], gsems[b]).wait()

  def fire_writes(t, b):
    base = wbase + t * CHUNK
    pltpu.async_copy(wb[b], out_hbm.at[pl.ds(base, CHUNK), pl.ds(0, WORD_DIM)], wsems[b])
    pltpu.async_copy(tb[b], out_hbm.at[pl.ds(base, CHUNK), pl.ds(128, SMALL_DIM)], wsems[b])
    pltpu.async_copy(p1b[b], out_hbm.at[pl.ds(base, CHUNK), pl.ds(144, SMALL_DIM)], wsems[b])
    pltpu.async_copy(p2b[b], out_hbm.at[pl.ds(base, CHUNK), pl.ds(160, SMALL_DIM)], wsems[b])

  def wait_writes(b):
    pltpu.make_async_copy(wb[b], out_hbm.at[pl.ds(wbase, CHUNK), pl.ds(0, WORD_DIM)], wsems[b]).wait()
    pltpu.make_async_copy(tb[b], out_hbm.at[pl.ds(wbase, CHUNK), pl.ds(128, SMALL_DIM)], wsems[b]).wait()
    pltpu.make_async_copy(p1b[b], out_hbm.at[pl.ds(wbase, CHUNK), pl.ds(144, SMALL_DIM)], wsems[b]).wait()
    pltpu.make_async_copy(p2b[b], out_hbm.at[pl.ds(wbase, CHUNK), pl.ds(160, SMALL_DIM)], wsems[b]).wait()


  for t in range(K):
    if t - D >= 0:
      wait_gathers((t - D) % K)
      fire_writes(t - D, (t - D) % K)
    fire_gathers(t, t % K)

  n_groups = (M - K) // K

  def group(g, _):
    for u in range(K):
      t = K + g * K + u
      bw = (K + u - D) % K
      wait_gathers(bw)
      fire_writes(t - D, bw)
      br = u % K
      wait_writes(br)
      fire_gathers(t, br)
    return ()

  lax.fori_loop(0, n_groups, group, ())

  for t in range(K + n_groups * K, M):
    bw = (t - D) % K
    wait_gathers(bw)
    fire_writes(t - D, bw)
    br = t % K
    wait_writes(br)
    fire_gathers(t, br)
  for t in range(M, M + D):
    bw = (t - D) % K
    wait_gathers(bw)
    fire_writes(t - D, bw)
  for i in range(K):
    b = (M - 1 - i) % K
    wait_writes(b)


@jax.jit
def _embed(idx4, word_tbl, tag_tbl, pos1_tbl, pos2_tbl):
  mesh = plsc.VectorSubcoreMesh(core_axis_name="c", subcore_axis_name="s")

  def body(idx4_hbm, wt, tt, p1t, p2t, out_hbm, *scratch):
    idxb = scratch[0:K]
    wb = scratch[K:2 * K]
    tb = scratch[2 * K:3 * K]
    p1b = scratch[3 * K:4 * K]
    p2b = scratch[4 * K:5 * K]
    gsems = scratch[5 * K:6 * K]
    wsems = scratch[6 * K:7 * K]
    _sc_body(idx4_hbm, wt, tt, p1t, p2t, out_hbm,
             idxb, wb, tb, p1b, p2b, gsems, wsems)

  scratch_types = (
      [pltpu.VMEM((4, CHUNK), jnp.int32) for _ in range(K)]
      + [pltpu.VMEM((CHUNK, WORD_DIM), jnp.float32) for _ in range(K)]
      + [pltpu.VMEM((CHUNK, SMALL_DIM), jnp.float32) for _ in range(K)]
      + [pltpu.VMEM((CHUNK, SMALL_DIM), jnp.float32) for _ in range(K)]
      + [pltpu.VMEM((CHUNK, SMALL_DIM), jnp.float32) for _ in range(K)]
      + [pltpu.SemaphoreType.DMA for _ in range(K)]
      + [pltpu.SemaphoreType.DMA for _ in range(K)]
  )
  f = pl.kernel(
      body,
      out_type=jax.ShapeDtypeStruct((N, OUT_DIM), jnp.float32),
      mesh=mesh,
      scratch_types=scratch_types,
      compiler_params=pltpu.CompilerParams(use_tc_tiling_on_sc=False),
  )
  return f(idx4, word_tbl, tag_tbl, pos1_tbl, pos2_tbl)


def kernel(word, tag, pos1, pos2, word_table, tag_table, pos1_table, pos2_table):
  idx4 = jnp.stack([
      word.reshape(N).astype(jnp.int32),
      tag.reshape(N).astype(jnp.int32),
      pos1.reshape(N).astype(jnp.int32),
      pos2.reshape(N).astype(jnp.int32),
  ])
  out = _embed(idx4, word_table, tag_table, pos1_table, pos2_table)
  return out.reshape(B, L, OUT_DIM)


# speedup vs baseline: 1.0041x; 1.0041x over previous
import jax
import jax.numpy as jnp
from jax import lax
from jax.experimental import pallas as pl
from jax.experimental.pallas import tpu as pltpu
from jax.experimental.pallas import tpu_sc as plsc

B = 1024
L = 200
N = B * L
WORD_DIM = 128
SMALL_DIM = 16
OUT_DIM = WORD_DIM + 3 * SMALL_DIM
PAD_DIM = 256

NC = 2
NS = 16
NW = NC * NS
N_PER_W = N // NW
CHUNK = 128
M = N_PER_W // CHUNK
K = 4
D = 2


def _sc_body(idx_hbm,
             word_tbl, tag_tbl, pos1_tbl, pos2_tbl,
             out_hbm,
             idxb, wb, tb, p1b, p2b, gsems, wsems):
  wid = lax.axis_index("s") * NC + lax.axis_index("c")
  wbase = wid * N_PER_W
  cbase = wid * M

  def fire_gathers(t, b):
    pltpu.sync_copy(idx_hbm.at[pl.ds((cbase + t) * 4 * CHUNK, 4 * CHUNK)], idxb[b])
    pltpu.async_copy(word_tbl.at[idxb[b].at[pl.ds(0, CHUNK)]], wb[b], gsems[b])
    pltpu.async_copy(tag_tbl.at[idxb[b].at[pl.ds(CHUNK, CHUNK)]], tb[b], gsems[b])
    pltpu.async_copy(pos1_tbl.at[idxb[b].at[pl.ds(2 * CHUNK, CHUNK)]], p1b[b], gsems[b])
    pltpu.async_copy(pos2_tbl.at[idxb[b].at[pl.ds(3 * CHUNK, CHUNK)]], p2b[b], gsems[b])

  def wait_gathers(b):
    pltpu.make_async_copy(word_tbl.at[idxb[b].at[pl.ds(0, CHUNK)]], wb[b], gsems[b]).wait()
    pltpu.make_async_copy(tag_tbl.at[idxb[b].at[pl.ds(CHUNK, CHUNK)]], tb[b], gsems[b]).wait()
    pltpu.make_async_copy(pos1_tbl.at[idxb[b].at[pl.ds(2 * CHUNK, CHUNK)]], p1b[b], gsems[b]).wait()
    pltpu.make_async_copy(pos2_tbl.at[idxb[b].at[pl.ds(3 * CHUNK, CHUNK)]], p2b[b], gsems[b]).wait()

  def fire_writes(t, b):
    base = wbase + t * CHUNK
    pltpu.async_copy(wb[b], out_hbm.at[pl.ds(base, CHUNK), pl.ds(0, WORD_DIM)], wsems[b])
    pltpu.async_copy(tb[b], out_hbm.at[pl.ds(base, CHUNK), pl.ds(128, SMALL_DIM)], wsems[b])
    pltpu.async_copy(p1b[b], out_hbm.at[pl.ds(base, CHUNK), pl.ds(144, SMALL_DIM)], wsems[b])
    pltpu.async_copy(p2b[b], out_hbm.at[pl.ds(base, CHUNK), pl.ds(160, SMALL_DIM)], wsems[b])

  def wait_writes(b):
    pltpu.make_async_copy(wb[b], out_hbm.at[pl.ds(wbase, CHUNK), pl.ds(0, WORD_DIM)], wsems[b]).wait()
    pltpu.make_async_copy(tb[b], out_hbm.at[pl.ds(wbase, CHUNK), pl.ds(128, SMALL_DIM)], wsems[b]).wait()
    pltpu.make_async_copy(p1b[b], out_hbm.at[pl.ds(wbase, CHUNK), pl.ds(144, SMALL_DIM)], wsems[b]).wait()
    pltpu.make_async_copy(p2b[b], out_hbm.at[pl.ds(wbase, CHUNK), pl.ds(160, SMALL_DIM)], wsems[b]).wait()


  for t in range(K):
    if t - D >= 0:
      wait_gathers((t - D) % K)
      fire_writes(t - D, (t - D) % K)
    fire_gathers(t, t % K)

  n_groups = (M - K) // K

  def group(g, _):
    for u in range(K):
      t = K + g * K + u
      bw = (K + u - D) % K
      wait_gathers(bw)
      fire_writes(t - D, bw)
      br = u % K
      wait_writes(br)
      fire_gathers(t, br)
    return ()

  lax.fori_loop(0, n_groups, group, ())

  for t in range(K + n_groups * K, M):
    bw = (t - D) % K
    wait_gathers(bw)
    fire_writes(t - D, bw)
    br = t % K
    wait_writes(br)
    fire_gathers(t, br)
  for t in range(M, M + D):
    bw = (t - D) % K
    wait_gathers(bw)
    fire_writes(t - D, bw)
  for i in range(K):
    wait_writes((M - 1 - i) % K)


def _sc_kernel_fn():
  mesh = plsc.VectorSubcoreMesh(core_axis_name="c", subcore_axis_name="s")

  def body(idx_hbm, wt, tt, p1t, p2t, out_hbm, *scratch):
    idxb = scratch[0:K]
    wb = scratch[K:2 * K]
    tb = scratch[2 * K:3 * K]
    p1b = scratch[3 * K:4 * K]
    p2b = scratch[4 * K:5 * K]
    gsems = scratch[5 * K:6 * K]
    wsems = scratch[6 * K:7 * K]
    _sc_body(idx_hbm, wt, tt, p1t, p2t, out_hbm,
             idxb, wb, tb, p1b, p2b, gsems, wsems)

  scratch_types = (
      [pltpu.VMEM((4 * CHUNK,), jnp.int32) for _ in range(K)]
      + [pltpu.VMEM((CHUNK, WORD_DIM), jnp.float32) for _ in range(K)]
      + [pltpu.VMEM((CHUNK, SMALL_DIM), jnp.float32) for _ in range(K)]
      + [pltpu.VMEM((CHUNK, SMALL_DIM), jnp.float32) for _ in range(K)]
      + [pltpu.VMEM((CHUNK, SMALL_DIM), jnp.float32) for _ in range(K)]
      + [pltpu.SemaphoreType.DMA for _ in range(K)]
      + [pltpu.SemaphoreType.DMA for _ in range(K)]
  )
  return pl.kernel(
      body,
      out_type=jax.ShapeDtypeStruct((N, PAD_DIM), jnp.float32),
      mesh=mesh,
      scratch_types=scratch_types,
      compiler_params=pltpu.CompilerParams(use_tc_tiling_on_sc=False),
  )


@jax.jit
def _embed(word, tag, pos1, pos2, word_tbl, tag_tbl, pos1_tbl, pos2_tbl):
  idx = jnp.stack([
      word.reshape(N // CHUNK, CHUNK).astype(jnp.int32),
      tag.reshape(N // CHUNK, CHUNK).astype(jnp.int32),
      pos1.reshape(N // CHUNK, CHUNK).astype(jnp.int32),
      pos2.reshape(N // CHUNK, CHUNK).astype(jnp.int32),
  ], axis=1).reshape(4 * N)
  out = _sc_kernel_fn()(idx, word_tbl, tag_tbl, pos1_tbl, pos2_tbl)
  return out[:, :OUT_DIM].reshape(B, L, OUT_DIM)


def kernel(word, tag, pos1, pos2, word_table, tag_table, pos1_table, pos2_table):
  return _embed(word, tag, pos1, pos2,
                word_table, tag_table, pos1_table, pos2_table)
